# baseline (speedup 1.0000x reference)
import functools

import jax
import jax.numpy as jnp
from jax import lax
from jax.experimental import pallas as pl
from jax.experimental.pallas import tpu as pltpu
from jax.experimental.pallas import tpu_sc as plsc

_NC = 2
_NS = 16
_IDS_PER_WORKER = 8


def _sc_gather(table, ids):
  b, c = ids.shape
  d = table.shape[1]
  per_row = c // _IDS_PER_WORKER
  mesh = plsc.VectorSubcoreMesh(
      core_axis_name="c", subcore_axis_name="s",
      num_cores=1, num_subcores=_NS)

  @functools.partial(
      pl.kernel,
      out_type=jax.ShapeDtypeStruct((b, c, d), jnp.float32),
      mesh=mesh,
      scratch_types=[
          pltpu.VMEM((_IDS_PER_WORKER,), jnp.int32),
          pltpu.VMEM((_IDS_PER_WORKER, d), jnp.float32),
          pltpu.SemaphoreType.DMA,
      ],
  )
  def gather_kernel(table_hbm, ids_hbm, out_hbm, idx_v, rows_v, sem):
    wid = lax.axis_index("s")
    row = wid // per_row
    col = (wid % per_row) * _IDS_PER_WORKER
    pltpu.sync_copy(ids_hbm.at[row, pl.ds(col, _IDS_PER_WORKER)], idx_v)
    pltpu.async_copy(table_hbm.at[idx_v], rows_v, sem).wait()
    pltpu.sync_copy(rows_v, out_hbm.at[row, pl.ds(col, _IDS_PER_WORKER)])

  return gather_kernel(table, ids)


_C_TILE = 8


def _add_body(x_ref, pe_ref, cp_ref, o_ref):
  o_ref[...] = (x_ref[...] + pe_ref[...][None, None]
                + cp_ref[...][:, :, None, :])


def kernel(x, channel_ids, pe_temporal, channel_table):
  b, c, t, d = x.shape
  cp = _sc_gather(channel_table, channel_ids.astype(jnp.int32))
  return pl.pallas_call(
      _add_body,
      grid=(b, c // _C_TILE),
      in_specs=[
          pl.BlockSpec((1, _C_TILE, t, d), lambda bi, i: (bi, i, 0, 0)),
          pl.BlockSpec((t, d), lambda bi, i: (0, 0)),
          pl.BlockSpec((1, _C_TILE, d), lambda bi, i: (bi, i, 0)),
      ],
      out_specs=pl.BlockSpec((1, _C_TILE, t, d), lambda bi, i: (bi, i, 0, 0)),
      out_shape=jax.ShapeDtypeStruct((b, c, t, d), jnp.float32),
  )(x, pe_temporal, cp)


# speedup vs baseline: 1.0009x; 1.0009x over previous
import functools

import jax
import jax.numpy as jnp
from jax import lax
from jax.experimental import pallas as pl
from jax.experimental.pallas import tpu as pltpu
from jax.experimental.pallas import tpu_sc as plsc

_NC = 2
_NS = 16
_IDS_PER_WORKER = 8


def _sc_gather(table, ids):
  b, c = ids.shape
  d = table.shape[1]
  per_row = c // _IDS_PER_WORKER
  mesh = plsc.VectorSubcoreMesh(
      core_axis_name="c", subcore_axis_name="s",
      num_cores=1, num_subcores=_NS)

  @functools.partial(
      pl.kernel,
      out_type=jax.ShapeDtypeStruct((b, c, d), jnp.float32),
      mesh=mesh,
      scratch_types=[
          pltpu.VMEM((_IDS_PER_WORKER,), jnp.int32),
          pltpu.VMEM((_IDS_PER_WORKER, d), jnp.float32),
          pltpu.SemaphoreType.DMA,
      ],
  )
  def gather_kernel(table_hbm, ids_hbm, out_hbm, idx_v, rows_v, sem):
    wid = lax.axis_index("s")
    row = wid // per_row
    col = (wid % per_row) * _IDS_PER_WORKER
    pltpu.sync_copy(ids_hbm.at[row, pl.ds(col, _IDS_PER_WORKER)], idx_v)
    pltpu.async_copy(table_hbm.at[idx_v], rows_v, sem).wait()
    pltpu.sync_copy(rows_v, out_hbm.at[row, pl.ds(col, _IDS_PER_WORKER)])

  return gather_kernel(table, ids)


_C_TILE = 4


def _add_body(x_ref, pe_ref, cp_ref, o_ref):
  bi = pl.program_id(0)
  i = pl.program_id(1)
  rows = cp_ref[pl.ds(bi, 1), pl.ds(i * _C_TILE, _C_TILE), :]
  o_ref[...] = (x_ref[...] + pe_ref[...][None, None]
                + rows[:, :, None, :])


def kernel(x, channel_ids, pe_temporal, channel_table):
  b, c, t, d = x.shape
  cp = _sc_gather(channel_table, channel_ids.astype(jnp.int32))
  return pl.pallas_call(
      _add_body,
      grid=(b, c // _C_TILE),
      in_specs=[
          pl.BlockSpec((1, _C_TILE, t, d), lambda bi, i: (bi, i, 0, 0)),
          pl.BlockSpec((t, d), lambda bi, i: (0, 0)),
          pl.BlockSpec((b, c, d), lambda bi, i: (0, 0, 0)),
      ],
      out_specs=pl.BlockSpec((1, _C_TILE, t, d), lambda bi, i: (bi, i, 0, 0)),
      out_shape=jax.ShapeDtypeStruct((b, c, t, d), jnp.float32),
  )(x, pe_temporal, cp)
